# Initial kernel scaffold; baseline (speedup 1.0000x reference)
import jax
import jax.numpy as jnp
from jax.experimental import pallas as pl


def kernel(var_val, var_type, boolean_table, pred_table):
    raise NotImplementedError("write your pallas kernel here")


# speedup vs baseline: 7.5617x; 7.5617x over previous
import functools

import jax
import jax.numpy as jnp
from jax import lax
from jax.experimental import pallas as pl
from jax.experimental.pallas import tpu as pltpu
from jax.experimental.pallas import tpu_sc as plsc

NC = 2
NS = 16
NW = NC * NS
LANES = 16
CHUNK = 1024


def _make_sc_kernel(N, V, D):
    per_w = N // NW
    n_chunks = per_w // CHUNK
    mesh = plsc.VectorSubcoreMesh(core_axis_name="c", subcore_axis_name="s")

    @functools.partial(
        pl.kernel,
        out_type=jax.ShapeDtypeStruct((N, D), jnp.float32),
        mesh=mesh,
        compiler_params=pltpu.CompilerParams(use_tc_tiling_on_sc=False),
        scratch_types=[
            pltpu.VMEM((CHUNK,), jnp.int32),
            pltpu.VMEM((CHUNK,), jnp.int32),
            pltpu.VMEM((CHUNK, D), jnp.float32),
            pltpu.VMEM((2, D), jnp.float32),
            pltpu.SemaphoreType.DMA,
        ],
    )
    def k(val_hbm, typ_hbm, bool_hbm, pred_hbm, out_hbm,
          idx_v, val_v, rows_v, bool_v, sem):
        wid = lax.axis_index("s") * NC + lax.axis_index("c")
        pltpu.sync_copy(bool_hbm, bool_v)
        b00 = bool_v[0, pl.ds(0, LANES)]
        b01 = bool_v[0, pl.ds(LANES, LANES)]
        d0 = bool_v[1, pl.ds(0, LANES)] - b00
        d1 = bool_v[1, pl.ds(LANES, LANES)] - b01

        def chunk_body(c, _):
            base = wid * per_w + c * CHUNK
            pltpu.sync_copy(typ_hbm.at[pl.ds(base, CHUNK)], idx_v)
            pltpu.sync_copy(val_hbm.at[pl.ds(base, CHUNK)], val_v)
            pltpu.async_copy(pred_hbm.at[idx_v], rows_v, sem).wait()

            def row_body(t, _):
                vval = val_v[pl.ds(t * LANES, LANES)].astype(jnp.float32)
                for j in range(LANES):
                    i = t * LANES + j
                    vf = vval[j]
                    r0 = rows_v[i, pl.ds(0, LANES)]
                    r1 = rows_v[i, pl.ds(LANES, LANES)]
                    rows_v[i, pl.ds(0, LANES)] = r0 * (b00 + vf * d0)
                    rows_v[i, pl.ds(LANES, LANES)] = r1 * (b01 + vf * d1)
                return ()

            lax.fori_loop(0, CHUNK // LANES, row_body, ())
            pltpu.sync_copy(rows_v, out_hbm.at[pl.ds(base, CHUNK)])
            return ()

        lax.fori_loop(0, n_chunks, chunk_body, ())

    return k


def kernel(var_val, var_type, boolean_table, pred_table):
    B, L = var_val.shape
    V, D = pred_table.shape
    N = B * L
    k = _make_sc_kernel(N, V, D)
    out = k(var_val.reshape(N), var_type.reshape(N), boolean_table, pred_table)
    return out.reshape(B, L, D)


# speedup vs baseline: 8.5947x; 1.1366x over previous
import functools

import jax
import jax.numpy as jnp
from jax import lax
from jax.experimental import pallas as pl
from jax.experimental.pallas import tpu as pltpu
from jax.experimental.pallas import tpu_sc as plsc

NC = 2
NS = 16
NW = NC * NS
LANES = 16
CHUNK = 800


def _make_sc_kernel(N, V, D):
    per_w = N // NW
    n_chunks = per_w // CHUNK
    mesh = plsc.VectorSubcoreMesh(core_axis_name="c", subcore_axis_name="s")

    @functools.partial(
        pl.kernel,
        out_type=jax.ShapeDtypeStruct((N, D), jnp.float32),
        mesh=mesh,
        compiler_params=pltpu.CompilerParams(use_tc_tiling_on_sc=False),
        scratch_types=[
            pltpu.VMEM((CHUNK,), jnp.int32),
            pltpu.VMEM((CHUNK,), jnp.int32),
            pltpu.VMEM((CHUNK,), jnp.int32),
            pltpu.VMEM((CHUNK,), jnp.int32),
            pltpu.VMEM((CHUNK, D), jnp.float32),
            pltpu.VMEM((CHUNK, D), jnp.float32),
            pltpu.VMEM((CHUNK, D), jnp.float32),
            pltpu.VMEM((CHUNK, D), jnp.float32),
            pltpu.VMEM((2, D), jnp.float32),
            pltpu.SemaphoreType.DMA,
            pltpu.SemaphoreType.DMA,
            pltpu.SemaphoreType.DMA,
            pltpu.SemaphoreType.DMA,
            pltpu.SemaphoreType.DMA,
            pltpu.SemaphoreType.DMA,
        ],
    )
    def k(val_hbm, typ_hbm, bool_hbm, pred_hbm, out_hbm,
          idx0, idx1, val0, val1, rin0, rin1, rout0, rout1, bool_v,
          sg0, sg1, si0, si1, so0, so1):
        idx = [idx0, idx1]
        val = [val0, val1]
        rin = [rin0, rin1]
        rout = [rout0, rout1]
        sg = [sg0, sg1]
        si = [si0, si1]
        so = [so0, so1]

        wid = lax.axis_index("s") * NC + lax.axis_index("c")
        w_base = wid * per_w
        pltpu.sync_copy(bool_hbm, bool_v)
        b00 = bool_v[0, pl.ds(0, LANES)]
        b01 = bool_v[0, pl.ds(LANES, LANES)]
        d0 = bool_v[1, pl.ds(0, LANES)] - b00
        d1 = bool_v[1, pl.ds(LANES, LANES)] - b01

        def fire_idx(h, b):
            base = w_base + h * CHUNK
            pltpu.async_copy(typ_hbm.at[pl.ds(base, CHUNK)], idx[b], si[b])

        def fire_val(h, b):
            base = w_base + h * CHUNK
            pltpu.async_copy(val_hbm.at[pl.ds(base, CHUNK)], val[b], si[b])

        def wait_idx(b):
            pltpu.make_async_copy(
                typ_hbm.at[pl.ds(0, CHUNK)], idx[b], si[b]).wait()
            pltpu.make_async_copy(
                val_hbm.at[pl.ds(0, CHUNK)], val[b], si[b]).wait()

        def step(g, b):
            pltpu.make_async_copy(pred_hbm.at[idx[b]], rin[b], sg[b]).wait()

            @pl.when(g + 2 < n_chunks)
            def _():
                fire_idx(g + 2, b)

            @pl.when(g + 1 < n_chunks)
            def _():
                wait_idx(b ^ 1)
                pltpu.async_copy(pred_hbm.at[idx[b ^ 1]], rin[b ^ 1],
                                 sg[b ^ 1])

            @pl.when(g >= 2)
            def _():
                pltpu.make_async_copy(
                    rout[b], out_hbm.at[pl.ds(0, CHUNK)], so[b]).wait()

            def row_body(t, _):
                vval = val[b][pl.ds(t * LANES, LANES)].astype(jnp.float32)
                for j in range(LANES):
                    i = t * LANES + j
                    vf = vval[j]
                    r0 = rin[b][i, pl.ds(0, LANES)]
                    r1 = rin[b][i, pl.ds(LANES, LANES)]
                    rout[b][i, pl.ds(0, LANES)] = r0 * (b00 + vf * d0)
                    rout[b][i, pl.ds(LANES, LANES)] = r1 * (b01 + vf * d1)
                return ()

            lax.fori_loop(0, CHUNK // LANES, row_body, ())

            @pl.when(g + 2 < n_chunks)
            def _():
                fire_val(g + 2, b)

            pltpu.async_copy(rout[b],
                             out_hbm.at[pl.ds(w_base + g * CHUNK, CHUNK)],
                             so[b])

        fire_idx(0, 0)
        fire_val(0, 0)
        fire_idx(1, 1)
        fire_val(1, 1)
        wait_idx(0)
        pltpu.async_copy(pred_hbm.at[idx[0]], rin[0], sg[0])

        def pair_body(cc, _):
            step(2 * cc, 0)
            step(2 * cc + 1, 1)
            return ()

        lax.fori_loop(0, n_chunks // 2, pair_body, ())

        pltpu.make_async_copy(rout[0], out_hbm.at[pl.ds(0, CHUNK)], so[0]).wait()
        pltpu.make_async_copy(rout[1], out_hbm.at[pl.ds(0, CHUNK)], so[1]).wait()

    return k


def kernel(var_val, var_type, boolean_table, pred_table):
    B, L = var_val.shape
    V, D = pred_table.shape
    N = B * L
    k = _make_sc_kernel(N, V, D)
    out = k(var_val.reshape(N), var_type.reshape(N), boolean_table, pred_table)
    return out.reshape(B, L, D)
